# baseline (speedup 1.0000x reference)
import functools

import jax
import jax.numpy as jnp
from jax.experimental import pallas as pl
from jax.experimental.pallas import tpu as pltpu

_N = 10000
_E = 2000
_D = 128
_NP = 10240
_NB = 2048
_GRID = _NP // _NB


def _intra_kernel(square_input, x_ref, incT_ref, out_ref, acc_ref, deg_ref):
    i = pl.program_id(0)
    v = x_ref[:]
    if square_input:
        v = v * v
    y = v.astype(jnp.bfloat16)
    part = jax.lax.dot_general(
        incT_ref[:], y, (((1,), (0,)), ((), ())),
        preferred_element_type=jnp.float32)
    dpart = jnp.sum(incT_ref[:].astype(jnp.float32), axis=1, keepdims=True)

    @pl.when(i == 0)
    def _init():
        acc_ref[:] = part
        deg_ref[:] = dpart

    @pl.when(i > 0)
    def _accum():
        acc_ref[:] += part
        deg_ref[:] += dpart

    @pl.when(i == _GRID - 1)
    def _finish():
        out_ref[:] = (
            acc_ref[:] / jnp.maximum(deg_ref[:], 1.0)
        ).astype(jnp.bfloat16)


def _inter_kernel(emit_squared, inc_ref, intra_ref, w_ref, out_ref):
    s2 = jax.lax.dot_general(
        inc_ref[:], intra_ref[:], (((1,), (0,)), ((), ())),
        preferred_element_type=jnp.float32)
    dv = jnp.sum(inc_ref[:].astype(jnp.float32), axis=1, keepdims=True)
    inter = jnp.sqrt(s2 / jnp.maximum(dv, 1.0))
    msg = jnp.dot(inter, w_ref[:], preferred_element_type=jnp.float32)
    act = jnp.maximum(msg, 0.0)
    if emit_squared:
        out_ref[:] = (act * act).astype(jnp.bfloat16)
    else:
        out_ref[:] = act


def _intra_call(x, incT, square_input):
    return pl.pallas_call(
        functools.partial(_intra_kernel, square_input),
        grid=(_GRID,),
        in_specs=[
            pl.BlockSpec((_NB, _D), lambda i: (i, 0)),
            pl.BlockSpec((_E, _NB), lambda i: (0, i)),
        ],
        out_specs=pl.BlockSpec((_E, _D), lambda i: (0, 0)),
        out_shape=jax.ShapeDtypeStruct((_E, _D), jnp.bfloat16),
        scratch_shapes=[
            pltpu.VMEM((_E, _D), jnp.float32),
            pltpu.VMEM((_E, 1), jnp.float32),
        ],
    )(x, incT)


def _inter_call(inc, intra, w, emit_squared):
    out_dtype = jnp.bfloat16 if emit_squared else jnp.float32
    return pl.pallas_call(
        functools.partial(_inter_kernel, emit_squared),
        grid=(_GRID,),
        in_specs=[
            pl.BlockSpec((_NB, _E), lambda i: (i, 0)),
            pl.BlockSpec((_E, _D), lambda i: (0, 0)),
            pl.BlockSpec((_D, _D), lambda i: (0, 0)),
        ],
        out_specs=pl.BlockSpec((_NB, _D), lambda i: (i, 0)),
        out_shape=jax.ShapeDtypeStruct((_NP, _D), out_dtype),
    )(inc, intra, w)


def kernel(x_0, incidence_1, W1, W2):
    inc_bf = incidence_1.astype(jnp.bfloat16)
    inc_p = jnp.pad(inc_bf, ((0, _NP - _N), (0, 0)))
    incT_p = jnp.pad(inc_bf.T, ((0, 0), (0, _NP - _N)))
    x_p = jnp.pad(x_0, ((0, _NP - _N), (0, 0)))

    intra1 = _intra_call(x_p, incT_p, square_input=True)
    y1 = _inter_call(inc_p, intra1, W1, emit_squared=True)
    intra2 = _intra_call(y1, incT_p, square_input=False)
    out = _inter_call(inc_p, intra2, W2, emit_squared=False)
    return out[:_N]


# speedup vs baseline: 2.1260x; 2.1260x over previous
import functools

import jax
import jax.numpy as jnp
from jax.experimental import pallas as pl
from jax.experimental.pallas import tpu as pltpu

_N = 10000
_E = 2000
_D = 128
_NB = 2000
_GRID = _N // _NB


def _intra_kernel_l1(x_ref, inc_ref, out_ref, dege_ref, acc_ref, dacc_ref):
    i = pl.program_id(0)
    v = x_ref[:]
    y = (v * v).astype(jnp.bfloat16)
    part = jax.lax.dot_general(
        inc_ref[:], y, (((0,), (0,)), ((), ())),
        preferred_element_type=jnp.float32)
    dpart = jnp.sum(inc_ref[:].astype(jnp.float32), axis=0, keepdims=True)

    @pl.when(i == 0)
    def _init():
        acc_ref[:] = part
        dacc_ref[:] = dpart

    @pl.when(i > 0)
    def _accum():
        acc_ref[:] += part
        dacc_ref[:] += dpart

    @pl.when(i == _GRID - 1)
    def _finish():
        deg = jnp.maximum(dacc_ref[:], 1.0).reshape(_E, 1)
        dege_ref[:] = deg
        out_ref[:] = (acc_ref[:] / deg).astype(jnp.bfloat16)


def _intra_kernel_l2(y_ref, inc_ref, dege_ref, out_ref, acc_ref):
    i = pl.program_id(0)
    part = jax.lax.dot_general(
        inc_ref[:], y_ref[:], (((0,), (0,)), ((), ())),
        preferred_element_type=jnp.float32)

    @pl.when(i == 0)
    def _init():
        acc_ref[:] = part

    @pl.when(i > 0)
    def _accum():
        acc_ref[:] += part

    @pl.when(i == _GRID - 1)
    def _finish():
        out_ref[:] = (acc_ref[:] / dege_ref[:]).astype(jnp.bfloat16)


def _inter_kernel_l1(inc_ref, intra_ref, w_ref, out_ref, degv_ref):
    s2 = jax.lax.dot_general(
        inc_ref[:], intra_ref[:], (((1,), (0,)), ((), ())),
        preferred_element_type=jnp.float32)
    dv = jnp.maximum(
        jnp.sum(inc_ref[:].astype(jnp.float32), axis=1, keepdims=True), 1.0)
    degv_ref[:] = dv
    inter = jnp.sqrt(s2 / dv)
    msg = jnp.dot(inter, w_ref[:], preferred_element_type=jnp.float32)
    act = jnp.maximum(msg, 0.0)
    out_ref[:] = (act * act).astype(jnp.bfloat16)


def _inter_kernel_l2(inc_ref, intra_ref, w_ref, degv_ref, out_ref):
    s2 = jax.lax.dot_general(
        inc_ref[:], intra_ref[:], (((1,), (0,)), ((), ())),
        preferred_element_type=jnp.float32)
    inter = jnp.sqrt(s2 / degv_ref[:])
    msg = jnp.dot(inter, w_ref[:], preferred_element_type=jnp.float32)
    out_ref[:] = jnp.maximum(msg, 0.0)


def kernel(x_0, incidence_1, W1, W2):
    inc_bf = incidence_1.astype(jnp.bfloat16)

    intra1, deg_e = pl.pallas_call(
        _intra_kernel_l1,
        grid=(_GRID,),
        in_specs=[
            pl.BlockSpec((_NB, _D), lambda i: (i, 0)),
            pl.BlockSpec((_NB, _E), lambda i: (i, 0)),
        ],
        out_specs=[
            pl.BlockSpec((_E, _D), lambda i: (0, 0)),
            pl.BlockSpec((_E, 1), lambda i: (0, 0)),
        ],
        out_shape=[
            jax.ShapeDtypeStruct((_E, _D), jnp.bfloat16),
            jax.ShapeDtypeStruct((_E, 1), jnp.float32),
        ],
        scratch_shapes=[
            pltpu.VMEM((_E, _D), jnp.float32),
            pltpu.VMEM((1, _E), jnp.float32),
        ],
    )(x_0, inc_bf)

    y1, deg_v = pl.pallas_call(
        _inter_kernel_l1,
        grid=(_GRID,),
        in_specs=[
            pl.BlockSpec((_NB, _E), lambda i: (i, 0)),
            pl.BlockSpec((_E, _D), lambda i: (0, 0)),
            pl.BlockSpec((_D, _D), lambda i: (0, 0)),
        ],
        out_specs=[
            pl.BlockSpec((_NB, _D), lambda i: (i, 0)),
            pl.BlockSpec((_NB, 1), lambda i: (i, 0)),
        ],
        out_shape=[
            jax.ShapeDtypeStruct((_N, _D), jnp.bfloat16),
            jax.ShapeDtypeStruct((_N, 1), jnp.float32),
        ],
    )(inc_bf, intra1, W1)

    intra2 = pl.pallas_call(
        _intra_kernel_l2,
        grid=(_GRID,),
        in_specs=[
            pl.BlockSpec((_NB, _D), lambda i: (i, 0)),
            pl.BlockSpec((_NB, _E), lambda i: (i, 0)),
            pl.BlockSpec((_E, 1), lambda i: (0, 0)),
        ],
        out_specs=pl.BlockSpec((_E, _D), lambda i: (0, 0)),
        out_shape=jax.ShapeDtypeStruct((_E, _D), jnp.bfloat16),
        scratch_shapes=[pltpu.VMEM((_E, _D), jnp.float32)],
    )(y1, inc_bf, deg_e)

    out = pl.pallas_call(
        _inter_kernel_l2,
        grid=(_GRID,),
        in_specs=[
            pl.BlockSpec((_NB, _E), lambda i: (i, 0)),
            pl.BlockSpec((_E, _D), lambda i: (0, 0)),
            pl.BlockSpec((_D, _D), lambda i: (0, 0)),
            pl.BlockSpec((_NB, 1), lambda i: (i, 0)),
        ],
        out_specs=pl.BlockSpec((_NB, _D), lambda i: (i, 0)),
        out_shape=jax.ShapeDtypeStruct((_N, _D), jnp.float32),
    )(inc_bf, intra2, W2, deg_v)

    return out
